# Initial kernel scaffold; baseline (speedup 1.0000x reference)
import jax
import jax.numpy as jnp
from jax.experimental import pallas as pl


def kernel(centroid, obj_conf):
    raise NotImplementedError("write your pallas kernel here")


# speedup vs baseline: 1.8061x; 1.8061x over previous
import jax
import jax.numpy as jnp
from jax.experimental import pallas as pl

_N = 512
_DIST2_THRESH = 0.2 * 0.2
_CONF_THRESH = 0.7


def _adj_kernel(in_ref, out_ref):
    b = pl.program_id(0)
    x = in_ref[0, 0:1, :]
    y = in_ref[0, 1:2, :]
    z = in_ref[0, 2:3, :]
    conf = in_ref[0, 3:4, :]

    xc = jnp.transpose(x)
    yc = jnp.transpose(y)
    zc = jnp.transpose(z)

    dx = xc - x
    dy = yc - y
    dz = zc - z
    d2 = dx * dx + dy * dy + dz * dz
    dist_ok = d2 > _DIST2_THRESH

    conf_ok = conf >= _CONF_THRESH
    conf_pair = jnp.transpose(conf_ok) & conf_ok

    rows = jax.lax.broadcasted_iota(jnp.int32, (_N, _N), 0)
    cols = jax.lax.broadcasted_iota(jnp.int32, (_N, _N), 1)
    off_diag = rows != cols
    use_dist = (b >= 2) & (rows >= 2)
    mask = conf_pair & off_diag & (dist_ok | jnp.logical_not(use_dist))
    out_ref[0] = mask.astype(jnp.float32)


def kernel(centroid, obj_conf):
    B, N, _ = centroid.shape
    packed = jnp.concatenate(
        [jnp.transpose(centroid, (0, 2, 1)), obj_conf[:, None, :]], axis=1
    )
    return pl.pallas_call(
        _adj_kernel,
        grid=(B,),
        in_specs=[pl.BlockSpec((1, 4, N), lambda b: (b, 0, 0))],
        out_specs=pl.BlockSpec((1, N, N), lambda b: (b, 0, 0)),
        out_shape=jax.ShapeDtypeStruct((B, N, N), jnp.float32),
    )(packed)
